# baseline (speedup 1.0000x reference)
import functools

import jax
import jax.numpy as jnp
from jax.experimental import pallas as pl
from jax.experimental.pallas import tpu as pltpu

N = 10000
F = 128
H = 32
FC = 512
BM = 200
NB = N // BM
BL = 400
NBL = N // BL
RB = 400
NRB = N // RB

_dot = functools.partial(
    jax.lax.dot_general,
    dimension_numbers=(((1,), (0,)), ((), ())),
    preferred_element_type=jnp.float32,
)


def _two_eye_block(i, bm):
    row = jax.lax.broadcasted_iota(jnp.int32, (bm, N), 0) + i * bm
    col = jax.lax.broadcasted_iota(jnp.int32, (bm, N), 1)
    return jnp.where(row == col, 2.0, 0.0)


def _deg_body(adj_ref, feat_ref, w1_ref, dinv_ref, z1_ref):
    i = pl.program_id(0)
    a = adj_ref[...] + _two_eye_block(i, BM)
    s = jnp.sum(a, axis=1, keepdims=True)
    dinv_ref[...] = jax.lax.rsqrt(s)
    z1_ref[...] = _dot(feat_ref[...], w1_ref[...])


def _l1_body(adj_ref, dinv_ref, drow_ref, z1_ref, b_ref,
             x1_ref, q_ref):
    i = pl.program_id(0)
    a = adj_ref[...] + _two_eye_block(i, BM)
    di = dinv_ref[...]
    ahat = (di * a) * drow_ref[...]
    q_ref[...] = ahat.astype(jnp.bfloat16)
    v = _dot(ahat, z1_ref[...]) + b_ref[...]
    x1_ref[...] = jnp.maximum(v, 0.0)


def _l23_body(q_ref, x1_ref, w2_ref, w3_ref, b_ref,
              out_ref, x_ref, z_ref, zhi_ref):
    l = pl.program_id(0)
    i = pl.program_id(1)

    @pl.when(i == 0)
    def _():
        @pl.when(l == 0)
        def _():
            z_ref[...] = _dot(x1_ref[...], w2_ref[...])

        @pl.when(l == 1)
        def _():
            z_ref[...] = _dot(x_ref[...], w3_ref[...])

        zhi_ref[...] = z_ref[...].astype(jnp.bfloat16)

    b = b_ref[pl.ds(l, 1), :]
    v = _dot(q_ref[...], zhi_ref[...]) + b

    @pl.when(l == 0)
    def _():
        x_ref[pl.ds(i * BL, BL), :] = jnp.maximum(v, 0.0)

    @pl.when(l == 1)
    def _():
        out_ref[...] = v


def _head_body(x_ref, g1_ref, bb1_ref, wf1_ref, bf1_ref, g2_ref, bb2_ref,
               wf2r_ref, bf2_ref, out_ref, y_ref, s2_ref, n1_ref, n2_ref):
    p = pl.program_id(0)
    i = pl.program_id(1)

    @pl.when((p == 0) & (i == 0))
    def _():
        x = x_ref[...]
        mu = jnp.mean(x, axis=0, keepdims=True)
        d = x - mu
        var = jnp.mean(d * d, axis=0, keepdims=True)
        n1_ref[0:1, :] = mu
        n1_ref[1:2, :] = jax.lax.rsqrt(var + 1e-5)
        s2_ref[...] = jnp.zeros_like(s2_ref)

    @pl.when(p == 0)
    def _phase_a():
        xb = x_ref[pl.ds(i * RB, RB), :]
        xn = (xb - n1_ref[0:1, :]) * n1_ref[1:2, :] * g1_ref[...] + bb1_ref[...]
        y = _dot(xn, wf1_ref[...]) + bf1_ref[...]
        y_ref[pl.ds(i * RB, RB), :] = y
        s2_ref[0:1, :] += jnp.sum(y, axis=0, keepdims=True)

    @pl.when(p == 1)
    def _phase_sq():
        @pl.when(i == 0)
        def _():
            s2_ref[1:2, :] = jnp.zeros_like(s2_ref[1:2, :])

        mu = s2_ref[0:1, :] * (1.0 / N)
        d = y_ref[pl.ds(i * RB, RB), :] - mu
        s2_ref[1:2, :] += jnp.sum(d * d, axis=0, keepdims=True)

    @pl.when(p == 2)
    def _phase_b():
        @pl.when(i == 0)
        def _():
            n2_ref[0:1, :] = s2_ref[0:1, :] * (1.0 / N)
            n2_ref[1:2, :] = jax.lax.rsqrt(s2_ref[1:2, :] * (1.0 / N) + 1e-5)

        y = y_ref[pl.ds(i * RB, RB), :]
        yn = (y - n2_ref[0:1, :]) * n2_ref[1:2, :] * g2_ref[...] + bb2_ref[...]
        act = jnp.where(yn >= 0.0, yn, 0.01 * yn)
        out_ref[...] = _dot(act, wf2r_ref[...]) + bf2_ref[...]


def kernel(adj, feat, W1, b1, W2, b2, W3, b3, bn1_g, bn1_b, Wf1, bf1,
           bn2_g, bn2_b, Wf2, bf2):
    adj = adj.reshape(N, N)
    feat = feat.reshape(N, F)

    dinv, z1 = pl.pallas_call(
        _deg_body,
        grid=(NB,),
        in_specs=[
            pl.BlockSpec((BM, N), lambda i: (i, 0)),
            pl.BlockSpec((BM, F), lambda i: (i, 0)),
            pl.BlockSpec((F, H), lambda i: (0, 0)),
        ],
        out_specs=(
            pl.BlockSpec((BM, 1), lambda i: (i, 0)),
            pl.BlockSpec((BM, H), lambda i: (i, 0)),
        ),
        out_shape=(
            jax.ShapeDtypeStruct((N, 1), jnp.float32),
            jax.ShapeDtypeStruct((N, H), jnp.float32),
        ),
    )(adj, feat, W1)

    dinv_row = dinv.reshape(1, N)

    x1, q = pl.pallas_call(
        _l1_body,
        grid=(NB,),
        in_specs=[
            pl.BlockSpec((BM, N), lambda i: (i, 0)),
            pl.BlockSpec((BM, 1), lambda i: (i, 0)),
            pl.BlockSpec((1, N), lambda i: (0, 0)),
            pl.BlockSpec((N, H), lambda i: (0, 0)),
            pl.BlockSpec((1, H), lambda i: (0, 0)),
        ],
        out_specs=(
            pl.BlockSpec((BM, H), lambda i: (i, 0)),
            pl.BlockSpec((BM, N), lambda i: (i, 0)),
        ),
        out_shape=(
            jax.ShapeDtypeStruct((N, H), jnp.float32),
            jax.ShapeDtypeStruct((N, N), jnp.bfloat16),
        ),
    )(adj, dinv, dinv_row, z1, b1.reshape(1, H))

    x3 = pl.pallas_call(
        _l23_body,
        grid=(2, NBL),
        in_specs=[
            pl.BlockSpec((BL, N), lambda l, i: (i, 0)),
            pl.BlockSpec((N, H), lambda l, i: (0, 0)),
            pl.BlockSpec((H, H), lambda l, i: (0, 0)),
            pl.BlockSpec((H, H), lambda l, i: (0, 0)),
            pl.BlockSpec((2, H), lambda l, i: (0, 0)),
        ],
        out_specs=pl.BlockSpec(
            (BL, H), lambda l, i: (jnp.where(l == 1, i, 0), 0)
        ),
        out_shape=jax.ShapeDtypeStruct((N, H), jnp.float32),
        scratch_shapes=[
            pltpu.VMEM((N, H), jnp.float32),
            pltpu.VMEM((N, H), jnp.float32),
            pltpu.VMEM((N, H), jnp.bfloat16),
        ],
    )(q, x1, W2, W3, jnp.stack([b2, b3], axis=0))

    out = pl.pallas_call(
        _head_body,
        grid=(3, NRB),
        in_specs=[
            pl.BlockSpec((N, H), lambda p, i: (0, 0)),
            pl.BlockSpec((1, H), lambda p, i: (0, 0)),
            pl.BlockSpec((1, H), lambda p, i: (0, 0)),
            pl.BlockSpec((H, FC), lambda p, i: (0, 0)),
            pl.BlockSpec((1, FC), lambda p, i: (0, 0)),
            pl.BlockSpec((1, FC), lambda p, i: (0, 0)),
            pl.BlockSpec((1, FC), lambda p, i: (0, 0)),
            pl.BlockSpec((FC, 1), lambda p, i: (0, 0)),
            pl.BlockSpec((1, 1), lambda p, i: (0, 0)),
        ],
        out_specs=pl.BlockSpec(
            (RB, 1), lambda p, i: (jnp.where(p == 2, i, 0), 0)
        ),
        out_shape=jax.ShapeDtypeStruct((N, 1), jnp.float32),
        scratch_shapes=[
            pltpu.VMEM((N, FC), jnp.float32),
            pltpu.VMEM((2, FC), jnp.float32),
            pltpu.VMEM((2, H), jnp.float32),
            pltpu.VMEM((2, FC), jnp.float32),
        ],
    )(
        x3,
        bn1_g.reshape(1, H), bn1_b.reshape(1, H),
        Wf1, bf1.reshape(1, FC),
        bn2_g.reshape(1, FC), bn2_b.reshape(1, FC),
        Wf2, bf2.reshape(1, 1),
    )
    return out
